# Initial kernel scaffold; baseline (speedup 1.0000x reference)
import jax
import jax.numpy as jnp
from jax.experimental import pallas as pl


def kernel(img_in, weight):
    raise NotImplementedError("write your pallas kernel here")


# speedup vs baseline: 28.0384x; 28.0384x over previous
import functools

import jax
import jax.numpy as jnp
from jax import lax
from jax.experimental import pallas as pl
from jax.experimental.pallas import tpu as pltpu
from jax.experimental.pallas import tpu_sc as plsc

UP = 4
Q = 16
L = 17
NW = 32
H_IN = 257
W_IN = 257
N_IMG = 4
H_OUT = 256
W_PAD = 264
ROWS_PER_WORKER = (N_IMG * H_OUT) // NW
CHUNK = 128
MAGIC = 12582912.0
INV_SCALE = 1.0 / 65536.0

W_ELEMS = (L ** 4) * (UP * UP)
QCHUNK = 5248
QCHUNKS_PER_WORKER = 8
W_PADDED = NW * QCHUNKS_PER_WORKER * QCHUNK
TABLE_ROWS = W_PADDED // 16

_CORNER_BITS = [[(m >> (3 - j)) & 1 for j in range(4)] for m in range(16)]
_CORNER_OFF = [b[0] * 4913 + b[1] * 289 + b[2] * 17 + b[3]
               for b in _CORNER_BITS]

_mesh = plsc.VectorSubcoreMesh(core_axis_name="c", subcore_axis_name="s")


def _worker_id():
    return lax.axis_index("s") * 2 + lax.axis_index("c")


@functools.partial(
    pl.kernel,
    out_type=jax.ShapeDtypeStruct((W_PADDED,), jnp.float32),
    mesh=_mesh,
    compiler_params=pltpu.CompilerParams(use_tc_tiling_on_sc=False,
                                         needs_layout_passes=False),
    scratch_types=[pltpu.VMEM((QCHUNK,), jnp.float32)],
)
def _quantize_lut(w_hbm, out_hbm, buf):
    wid = _worker_id()
    base = wid * (QCHUNKS_PER_WORKER * QCHUNK)

    def chunk_body(ci, _):
        off = base + ci * QCHUNK
        pltpu.sync_copy(w_hbm.at[pl.ds(off, QCHUNK)], buf)

        def vec_body(v, _):
            x = buf[pl.ds(v * 16, 16)]
            y = x * 127.0
            r = (y + MAGIC) - MAGIC
            r = jnp.minimum(jnp.maximum(r, -127.0), 127.0)
            buf[pl.ds(v * 16, 16)] = r
            return 0

        lax.fori_loop(0, QCHUNK // 16, vec_body, 0)
        pltpu.sync_copy(buf, out_hbm.at[pl.ds(off, QCHUNK)])
        return 0

    lax.fori_loop(0, QCHUNKS_PER_WORKER, chunk_body, 0)


@functools.partial(
    pl.kernel,
    out_type=jax.ShapeDtypeStruct((N_IMG * H_OUT * UP, H_OUT * UP),
                                  jnp.float32),
    mesh=_mesh,
    compiler_params=pltpu.CompilerParams(use_tc_tiling_on_sc=False,
                                         needs_layout_passes=False),
    scratch_types=[
        pltpu.VMEM((2, W_PAD), jnp.int32),
        pltpu.VMEM((16, CHUNK), jnp.int32),
        pltpu.VMEM((16, CHUNK), jnp.float32),
        pltpu.VMEM((16 * CHUNK, 16), jnp.float32),
        pltpu.VMEM((UP, H_OUT * UP), jnp.float32),
        pltpu.SemaphoreType.DMA,
    ],
)
def _mulut_main(table_hbm, img_hbm, out_hbm, img_v, idx_v, coef_v, rows_v,
                out_v, sem):
    wid = _worker_id()
    iota = lax.broadcasted_iota(jnp.int32, (16,), 0)

    def row_body(t, _):
        gi = wid * ROWS_PER_WORKER + t
        img_idx = gi >> 8
        i = gi & 255
        src_row = img_idx * H_IN + i
        pltpu.sync_copy(img_hbm.at[pl.ds(src_row, 2)], img_v)

        for ch in range(2):
            p0 = ch * CHUNK

            def gen_body(g, _):
                o = p0 + g * 16
                a = img_v[0, pl.ds(o, 16)]
                b = img_v[0, pl.ds(o + 1, 16)]
                c = img_v[1, pl.ds(o, 16)]
                d = img_v[1, pl.ds(o + 1, 16)]
                base = ((a >> 4) * 4913 + (b >> 4) * 289
                        + (c >> 4) * 17 + (d >> 4))
                fa = (a & 15).astype(jnp.float32)
                fb = (b & 15).astype(jnp.float32)
                fc = (c & 15).astype(jnp.float32)
                fd = (d & 15).astype(jnp.float32)
                ta = (16.0 - fa, fa)
                tb = (16.0 - fb, fb)
                tc = (16.0 - fc, fc)
                td = (16.0 - fd, fd)
                pab = [[ta[x] * tb[y] for y in range(2)] for x in range(2)]
                pcd = [[tc[x] * td[y] for y in range(2)] for x in range(2)]
                col = pl.ds(g * 16, 16)
                for m in range(16):
                    b0, b1, b2, b3 = _CORNER_BITS[m]
                    idx_v[m, col] = base + _CORNER_OFF[m]
                    coef_v[m, col] = pab[b0][b1] * pcd[b2][b3]
                return 0

            lax.fori_loop(0, CHUNK // 16, gen_body, 0)

            copies = [
                pltpu.async_copy(table_hbm.at[idx_v.at[m]],
                                 rows_v.at[pl.ds(m * CHUNK, CHUNK)], sem)
                for m in range(16)
            ]
            for cp in copies:
                cp.wait()

            def acc_body(g, _):
                rbase = g * 16 + iota
                acc = [None] * 16
                for m in range(16):
                    cvec = coef_v[m, pl.ds(g * 16, 16)]
                    ridx = rbase + m * CHUNK
                    for k in range(16):
                        kcol = jnp.full((16,), k, jnp.int32)
                        val = plsc.load_gather(rows_v, [ridx, kcol])
                        if m == 0:
                            acc[k] = cvec * val
                        else:
                            acc[k] = acc[k] + cvec * val
                ocol = (rbase + p0) * UP
                for k in range(16):
                    orow = jnp.full((16,), k // UP, jnp.int32)
                    plsc.store_scatter(out_v, [orow, ocol + (k % UP)],
                                       acc[k] * INV_SCALE)
                return 0

            lax.fori_loop(0, CHUNK // 16, acc_body, 0)

        pltpu.sync_copy(out_v, out_hbm.at[pl.ds(gi * UP, UP)])
        return 0

    lax.fori_loop(0, ROWS_PER_WORKER, row_body, 0)


def kernel(img_in, weight):
    B, C, H, W = img_in.shape
    img2 = img_in.reshape(B * C * H, W)
    img2 = jnp.pad(img2, ((0, 0), (0, W_PAD - W)))
    wflat = weight.reshape(-1)
    wpad = jnp.pad(wflat, (0, W_PADDED - wflat.shape[0]))
    wq = _quantize_lut(wpad)
    table = wq.reshape(TABLE_ROWS, 16)
    out = _mulut_main(table, img2)
    return out.reshape(B, C, H_OUT * UP, H_OUT * UP)


# speedup vs baseline: 40.8632x; 1.4574x over previous
import functools

import jax
import jax.numpy as jnp
from jax import lax
from jax.experimental import pallas as pl
from jax.experimental.pallas import tpu as pltpu
from jax.experimental.pallas import tpu_sc as plsc

UP = 4
Q = 16
L = 17
NW = 32
H_IN = 257
W_IN = 257
N_IMG = 4
H_OUT = 256
W_PAD = 264
ROWS_PER_WORKER = (N_IMG * H_OUT) // NW
CHUNK = 128
MAGIC = 12582912.0
INV_SCALE = 1.0 / 65536.0

W_ELEMS = (L ** 4) * (UP * UP)
QCHUNK = 5248
QCHUNKS_PER_WORKER = 8
W_PADDED = NW * QCHUNKS_PER_WORKER * QCHUNK
TABLE_ROWS = W_PADDED // 16

_CORNER_BITS = [[(m >> (3 - j)) & 1 for j in range(4)] for m in range(16)]
_CORNER_OFF = [b[0] * 4913 + b[1] * 289 + b[2] * 17 + b[3]
               for b in _CORNER_BITS]

_mesh = plsc.VectorSubcoreMesh(core_axis_name="c", subcore_axis_name="s")


def _worker_id():
    return lax.axis_index("s") * 2 + lax.axis_index("c")


@functools.partial(
    pl.kernel,
    out_type=jax.ShapeDtypeStruct((W_PADDED,), jnp.float32),
    mesh=_mesh,
    compiler_params=pltpu.CompilerParams(use_tc_tiling_on_sc=False,
                                         needs_layout_passes=False),
    scratch_types=[pltpu.VMEM((QCHUNK,), jnp.float32)],
)
def _quantize_lut(w_hbm, out_hbm, buf):
    wid = _worker_id()
    base = wid * (QCHUNKS_PER_WORKER * QCHUNK)

    def chunk_body(ci, _):
        off = base + ci * QCHUNK
        pltpu.sync_copy(w_hbm.at[pl.ds(off, QCHUNK)], buf)

        def vec_body(v, _):
            x = buf[pl.ds(v * 16, 16)]
            y = x * 127.0
            r = (y + MAGIC) - MAGIC
            r = jnp.minimum(jnp.maximum(r, -127.0), 127.0)
            buf[pl.ds(v * 16, 16)] = r
            return 0

        lax.fori_loop(0, QCHUNK // 16, vec_body, 0)
        pltpu.sync_copy(buf, out_hbm.at[pl.ds(off, QCHUNK)])
        return 0

    lax.fori_loop(0, QCHUNKS_PER_WORKER, chunk_body, 0)


@functools.partial(
    pl.kernel,
    out_type=jax.ShapeDtypeStruct((N_IMG * H_OUT * UP, H_OUT * UP),
                                  jnp.float32),
    mesh=_mesh,
    compiler_params=pltpu.CompilerParams(use_tc_tiling_on_sc=False,
                                         needs_layout_passes=False),
    scratch_types=[
        pltpu.VMEM((2, W_PAD), jnp.int32),
        pltpu.VMEM((2 * 16, CHUNK), jnp.int32),
        pltpu.VMEM((2 * 16, CHUNK), jnp.float32),
        pltpu.VMEM((2 * 16 * CHUNK, 16), jnp.float32),
        pltpu.VMEM((UP, H_OUT * UP), jnp.float32),
        pltpu.SemaphoreType.DMA,
        pltpu.SemaphoreType.DMA,
    ],
)
def _mulut_main(table_hbm, img_hbm, out_hbm, img_v, idx_v, coef_v, rows_v,
                out_v, sem0, sem1):
    wid = _worker_id()
    iota = lax.broadcasted_iota(jnp.int32, (16,), 0)
    sems = (sem0, sem1)
    NROWS = 16 * CHUNK

    def load_img(t):
        src_row = (t >> 8) * H_IN + (t & 255)
        pltpu.sync_copy(img_hbm.at[pl.ds(src_row, 2)], img_v)

    def gen(ch, buf):
        p0 = ch * CHUNK

        def gen_body(g, _):
            o = p0 + g * 16
            a = img_v[0, pl.ds(o, 16)]
            b = img_v[0, pl.ds(o + 1, 16)]
            c = img_v[1, pl.ds(o, 16)]
            d = img_v[1, pl.ds(o + 1, 16)]
            base = ((a >> 4) * 4913 + (b >> 4) * 289
                    + (c >> 4) * 17 + (d >> 4))
            fa = (a & 15).astype(jnp.float32)
            fb = (b & 15).astype(jnp.float32)
            fc = (c & 15).astype(jnp.float32)
            fd = (d & 15).astype(jnp.float32)
            ta = (16.0 - fa, fa)
            tb = (16.0 - fb, fb)
            tc = (16.0 - fc, fc)
            td = (16.0 - fd, fd)
            pab = [[ta[x] * tb[y] for y in range(2)] for x in range(2)]
            pcd = [[tc[x] * td[y] for y in range(2)] for x in range(2)]
            col = pl.ds(g * 16, 16)
            for m in range(16):
                b0, b1, b2, b3 = _CORNER_BITS[m]
                idx_v[buf * 16 + m, col] = base + _CORNER_OFF[m]
                coef_v[buf * 16 + m, col] = pab[b0][b1] * pcd[b2][b3]
            return 0

        lax.fori_loop(0, CHUNK // 16, gen_body, 0)

    def fire(buf):
        for m in range(16):
            pltpu.async_copy(
                table_hbm.at[idx_v.at[buf * 16 + m]],
                rows_v.at[pl.ds(buf * NROWS + m * CHUNK, CHUNK)], sems[buf])

    def drain(buf):
        pltpu.make_async_copy(table_hbm.at[pl.ds(0, NROWS)],
                              rows_v.at[pl.ds(buf * NROWS, NROWS)],
                              sems[buf]).wait()

    def acc(ch, buf):
        p0 = ch * CHUNK

        def acc_body(g, _):
            rbase = buf * NROWS + g * 16 + iota
            accs = [None] * 16
            for m in range(16):
                cvec = coef_v[buf * 16 + m, pl.ds(g * 16, 16)]
                ridx = rbase + m * CHUNK
                for k in range(16):
                    kcol = jnp.full((16,), k, jnp.int32)
                    val = plsc.load_gather(rows_v, [ridx, kcol])
                    if m == 0:
                        accs[k] = cvec * val
                    else:
                        accs[k] = accs[k] + cvec * val
            ocol = (g * 16 + iota + p0) * UP
            for k in range(16):
                orow = jnp.full((16,), k // UP, jnp.int32)
                plsc.store_scatter(out_v, [orow, ocol + (k % UP)],
                                   accs[k] * INV_SCALE)
            return 0

        lax.fori_loop(0, CHUNK // 16, acc_body, 0)

    t0 = wid * ROWS_PER_WORKER
    load_img(t0)
    gen(0, 0)
    fire(0)
    gen(1, 1)
    fire(1)

    def row_body(i, _):
        t = t0 + i
        drain(0)
        acc(0, 0)

        @pl.when(i < ROWS_PER_WORKER - 1)
        def _():
            load_img(t + 1)
            gen(0, 0)
            fire(0)

        drain(1)
        acc(1, 1)

        @pl.when(i < ROWS_PER_WORKER - 1)
        def _():
            gen(1, 1)
            fire(1)

        pltpu.sync_copy(out_v, out_hbm.at[pl.ds(t * UP, UP)])
        return 0

    lax.fori_loop(0, ROWS_PER_WORKER, row_body, 0)


def kernel(img_in, weight):
    B, C, H, W = img_in.shape
    img2 = img_in.reshape(B * C * H, W)
    img2 = jnp.pad(img2, ((0, 0), (0, W_PAD - W)))
    wflat = weight.reshape(-1)
    wpad = jnp.pad(wflat, (0, W_PADDED - wflat.shape[0]))
    wq = _quantize_lut(wpad)
    table = wq.reshape(TABLE_ROWS, 16)
    out = _mulut_main(table, img2)
    return out.reshape(B, C, H_OUT * UP, H_OUT * UP)


# speedup vs baseline: 49.5014x; 1.2114x over previous
import functools

import jax
import jax.numpy as jnp
from jax import lax
from jax.experimental import pallas as pl
from jax.experimental.pallas import tpu as pltpu
from jax.experimental.pallas import tpu_sc as plsc

UP = 4
H_IN = 257
N_IMG = 4
H_OUT = 256
W_PAD = 264
ROWS_PER_WORKER = (N_IMG * H_OUT) // 32
CHUNK = 128
MAGIC = 12582912.0
INV_SCALE = 1.0 / 65536.0

TABLE_ROWS = 83968
QCHUNK_ROWS = 656
QIN = (QCHUNK_ROWS + 3) * 16
W_QPAD = (TABLE_ROWS + 38) * 16

_PAIR_OFF = [0, 17, 289, 306, 4913, 4930, 5202, 5219]

_mesh = plsc.VectorSubcoreMesh(core_axis_name="c", subcore_axis_name="s")
_params = pltpu.CompilerParams(use_tc_tiling_on_sc=False,
                               needs_layout_passes=False)


def _round_clip(x):
    y = x * 127.0
    r = (y + MAGIC) - MAGIC
    return jnp.minimum(jnp.maximum(r, -127.0), 127.0)


@functools.partial(
    pl.kernel,
    out_type=(jax.ShapeDtypeStruct((N_IMG * H_OUT * UP, H_OUT * UP),
                                   jnp.float32),
              jax.ShapeDtypeStruct((2, TABLE_ROWS, 16), jnp.int32)),
    mesh=_mesh,
    compiler_params=_params,
    scratch_types=[
        pltpu.VMEM((2, QIN), jnp.float32),
        pltpu.VMEM((2, QCHUNK_ROWS, 16), jnp.int32),
        pltpu.VMEM((2, W_PAD), jnp.int32),
        pltpu.VMEM((2 * 8, CHUNK), jnp.int32),
        pltpu.VMEM((2 * 16, CHUNK), jnp.float32),
        pltpu.VMEM((2 * 8 * CHUNK, 16), jnp.int32),
        pltpu.VMEM((UP, H_OUT * UP), jnp.float32),
        pltpu.SemaphoreType.DMA,
        pltpu.SemaphoreType.DMA,
        pltpu.SemaphoreType.DMA,
        pltpu.SemaphoreType.DMA,
        pltpu.SemaphoreType.DMA,
        pltpu.SemaphoreType.DMA,
        pltpu.SemaphoreType.DMA,
        pltpu.SemaphoreType.DMA,
    ],
)
def _mulut(w_hbm, img_hbm, out_hbm, tab_hbm, rbuf, obuf, img_v, idx_v,
           coef_v, rows_v, out_v, sem0, sem1, qsl0, qsl1, qss0, qss1,
           sem_img, sem_out):
    ci = lax.axis_index("c")
    sid = lax.axis_index("s")
    wid = sid * 2 + ci
    iota = lax.broadcasted_iota(jnp.int32, (16,), 0)
    mask_hi = jnp.full((16,), -65536, jnp.int32)

    tile_row0 = sid * (8 * QCHUNK_ROWS)
    qsl = (qsl0, qsl1)
    qss = (qss0, qss1)

    def q_load(c, b):
        lo = tile_row0 + c * QCHUNK_ROWS
        return pltpu.async_copy(w_hbm.at[pl.ds(lo * 16, QIN)], rbuf.at[b],
                                qsl[b])

    def q_store(c, b):
        lo = tile_row0 + c * QCHUNK_ROWS
        return pltpu.make_async_copy(
            obuf.at[b], tab_hbm.at[ci, pl.ds(lo, QCHUNK_ROWS), :], qss[b])

    def q_compute(b):
        def rc_body(v, _):
            rbuf[b, pl.ds(v * 16, 16)] = _round_clip(rbuf[b, pl.ds(v * 16, 16)])
            return 0

        lax.fori_loop(0, QCHUNK_ROWS + 1, rc_body, 0)

        def pack_body(j, _):
            b0 = plsc.bitcast(rbuf[b, pl.ds(j * 16, 16)], jnp.int32)
            b1 = plsc.bitcast(rbuf[b, pl.ds(j * 16 + 16, 16)], jnp.int32)
            obuf[b, j, :] = (
                lax.shift_right_logical(b0, 16) | (b1 & mask_hi))
            return 0

        lax.fori_loop(0, QCHUNK_ROWS, pack_body, 0)

    ld0 = q_load(0, 0)
    ld1 = q_load(1, 1)
    for it in range(4):
        c0, c1 = 2 * it, 2 * it + 1
        ld0.wait()
        if it > 0:
            q_store(c0 - 2, 0).wait()
        q_compute(0)
        q_store(c0, 0).start()
        if it < 3:
            ld0 = q_load(c0 + 2, 0)
        ld1.wait()
        if it > 0:
            q_store(c1 - 2, 1).wait()
        q_compute(1)
        q_store(c1, 1).start()
        if it < 3:
            ld1 = q_load(c1 + 2, 1)
    q_store(6, 0).wait()
    q_store(7, 1).wait()

    plsc.subcore_barrier()

    table = tab_hbm.at[ci]
    sems = (sem0, sem1)
    NROWS = 8 * CHUNK

    def img_copy(t):
        src_row = (t >> 8) * H_IN + (t & 255)
        return pltpu.make_async_copy(img_hbm.at[pl.ds(src_row, 2)], img_v,
                                     sem_img)

    def out_copy(t):
        return pltpu.make_async_copy(out_v, out_hbm.at[pl.ds(t * UP, UP)],
                                     sem_out)

    def gen(ch, buf):
        p0 = ch * CHUNK

        def gen_body(g, _):
            o = p0 + g * 16
            a = img_v[0, pl.ds(o, 16)]
            b = img_v[0, pl.ds(o + 1, 16)]
            c = img_v[1, pl.ds(o, 16)]
            d = img_v[1, pl.ds(o + 1, 16)]
            base = ((a >> 4) * 4913 + (b >> 4) * 289
                    + (c >> 4) * 17 + (d >> 4))
            fa = (a & 15).astype(jnp.float32)
            fb = (b & 15).astype(jnp.float32)
            fc = (c & 15).astype(jnp.float32)
            fd = (d & 15).astype(jnp.float32)
            ta = (16.0 - fa, fa)
            tb = (16.0 - fb, fb)
            tc = (16.0 - fc, fc)
            td = (16.0 - fd, fd)
            pab = [[ta[x] * tb[y] for y in range(2)] for x in range(2)]
            pcd = [[tc[x] * td[y] for y in range(2)] for x in range(2)]
            col = pl.ds(g * 16, 16)
            for p in range(8):
                idx_v[buf * 8 + p, col] = base + _PAIR_OFF[p]
            for m in range(16):
                b0, b1, b2, b3 = (m >> 3) & 1, (m >> 2) & 1, (m >> 1) & 1, m & 1
                coef_v[buf * 16 + m, col] = pab[b0][b1] * pcd[b2][b3]
            return 0

        lax.fori_loop(0, CHUNK // 16, gen_body, 0)

    def fire(buf):
        for p in range(8):
            pltpu.async_copy(
                table.at[idx_v.at[buf * 8 + p]],
                rows_v.at[pl.ds(buf * NROWS + p * CHUNK, CHUNK)],
                sems[buf])

    def drain(buf):
        pltpu.make_async_copy(table.at[pl.ds(0, NROWS)],
                              rows_v.at[pl.ds(buf * NROWS, NROWS)],
                              sems[buf]).wait()

    dnums = lax.GatherDimensionNumbers(offset_dims=(),
                                       collapsed_slice_dims=(0,),
                                       start_index_map=(0,))

    def bcast(v, m):
        return lax.gather(v, jnp.full((16, 1), m, jnp.int32), dnums,
                          (1,), mode=lax.GatherScatterMode.PROMISE_IN_BOUNDS)

    def acc(ch, buf):
        p0 = ch * CHUNK

        def acc_body(jg, _):
            col = pl.ds(jg * 16, 16)
            cl = [coef_v[buf * 16 + m, col] for m in range(16)]
            orow = lax.shift_right_logical(iota, 2)
            for u in range(16):
                j = jg * 16 + u
                accv = None
                for p in range(8):
                    xr = rows_v[buf * NROWS + p * CHUNK + j, :]
                    vlo = plsc.bitcast(lax.shift_left(xr, 16), jnp.float32)
                    vhi = plsc.bitcast(xr & mask_hi, jnp.float32)
                    t = (bcast(cl[2 * p], u) * vlo
                         + bcast(cl[2 * p + 1], u) * vhi)
                    accv = t if p == 0 else accv + t
                plsc.store_scatter(
                    out_v, [orow, (p0 + j) * UP + (iota & 3)],
                    accv * INV_SCALE)
            return 0

        lax.fori_loop(0, CHUNK // 16, acc_body, 0)

    t0 = wid * ROWS_PER_WORKER
    img_copy(t0).start()
    img_copy(t0).wait()
    gen(0, 0)
    fire(0)
    gen(1, 1)
    fire(1)

    def row_body(i, _):
        t = t0 + i

        @pl.when(i < ROWS_PER_WORKER - 1)
        def _():
            img_copy(t + 1).start()

        drain(0)

        @pl.when(i > 0)
        def _():
            out_copy(t - 1).wait()

        acc(0, 0)

        @pl.when(i < ROWS_PER_WORKER - 1)
        def _():
            img_copy(t + 1).wait()
            gen(0, 0)
            fire(0)

        drain(1)
        acc(1, 1)

        @pl.when(i < ROWS_PER_WORKER - 1)
        def _():
            gen(1, 1)
            fire(1)

        out_copy(t).start()
        return 0

    lax.fori_loop(0, ROWS_PER_WORKER, row_body, 0)
    out_copy(t0 + ROWS_PER_WORKER - 1).wait()


def kernel(img_in, weight):
    B, C, H, W = img_in.shape
    img2 = img_in.reshape(B * C * H, W)
    img2 = jnp.pad(img2, ((0, 0), (0, W_PAD - W)))
    wflat = weight.reshape(-1)
    wpad = jnp.pad(wflat, (0, W_QPAD - wflat.shape[0]))
    out, _ = _mulut(wpad, img2)
    return out.reshape(B, C, H_OUT * UP, H_OUT * UP)


# speedup vs baseline: 56.5955x; 1.1433x over previous
import functools

import jax
import jax.numpy as jnp
from jax import lax
from jax.experimental import pallas as pl
from jax.experimental.pallas import tpu as pltpu
from jax.experimental.pallas import tpu_sc as plsc

UP = 4
NW = 32
H_IN = 257
N_IMG = 4
H_OUT = 256
W_PAD = 264
ROWS_PER_WORKER = (N_IMG * H_OUT) // NW
CHUNK = 128
MAGIC = 12582912.0
INV_SCALE = 1.0 / 65536.0

TABLE_ROWS = 83968
QCHUNK_ROWS = 328
QOVER = 19
W_QPAD = (TABLE_ROWS + 2 * QOVER) * 16

_PAIR_OFF = [0, 17, 289, 306, 4913, 4930, 5202, 5219]

_mesh = plsc.VectorSubcoreMesh(core_axis_name="c", subcore_axis_name="s")
_params = pltpu.CompilerParams(use_tc_tiling_on_sc=False,
                               needs_layout_passes=False)


def _worker_id():
    return lax.axis_index("s") * 2 + lax.axis_index("c")


def _round_clip(x):
    y = x * 127.0
    r = (y + MAGIC) - MAGIC
    return jnp.minimum(jnp.maximum(r, -127.0), 127.0)


@functools.partial(
    pl.kernel,
    out_type=jax.ShapeDtypeStruct((TABLE_ROWS * 16,), jnp.int32),
    mesh=_mesh,
    compiler_params=_params,
    scratch_types=[
        pltpu.VMEM(((QCHUNK_ROWS + QOVER) * 16,), jnp.float32),
        pltpu.VMEM(((QCHUNK_ROWS + QOVER) * 16,), jnp.float32),
        pltpu.VMEM((QCHUNK_ROWS * 16,), jnp.int32),
    ],
)
def _quantize_lut(w_hbm, out_hbm, rbuf, qbuf, obuf):
    wid = _worker_id()
    base_row = wid * (8 * QCHUNK_ROWS)
    n_in = (QCHUNK_ROWS + QOVER) * 16
    mask_hi = jnp.full((16,), -65536, jnp.int32)

    def chunk_body(ci, _):
        lo = base_row + ci * QCHUNK_ROWS
        pltpu.sync_copy(w_hbm.at[pl.ds(lo * 16, n_in)], rbuf)

        def rc_body(v, _):
            qbuf[pl.ds(v * 16, 16)] = _round_clip(rbuf[pl.ds(v * 16, 16)])
            return 0

        lax.fori_loop(0, QCHUNK_ROWS + QOVER, rc_body, 0)

        def pack_body(j, _):
            b0 = plsc.bitcast(qbuf[pl.ds(j * 16, 16)], jnp.int32)
            b1 = plsc.bitcast(qbuf[pl.ds(j * 16 + 16, 16)], jnp.int32)
            obuf[pl.ds(j * 16, 16)] = (
                lax.shift_right_logical(b0, 16) | (b1 & mask_hi))
            return 0

        lax.fori_loop(0, QCHUNK_ROWS, pack_body, 0)
        pltpu.sync_copy(obuf, out_hbm.at[pl.ds(lo * 16, QCHUNK_ROWS * 16)])
        return 0

    lax.fori_loop(0, 8, chunk_body, 0)


@functools.partial(
    pl.kernel,
    out_type=jax.ShapeDtypeStruct((N_IMG * H_OUT * UP, H_OUT * UP),
                                  jnp.float32),
    mesh=_mesh,
    compiler_params=_params,
    scratch_types=[
        pltpu.VMEM((2, W_PAD), jnp.int32),
        pltpu.VMEM((2 * 8, CHUNK), jnp.int32),
        pltpu.VMEM((2 * 16, CHUNK), jnp.float32),
        pltpu.VMEM((2 * 8 * CHUNK, 16), jnp.int32),
        pltpu.VMEM((UP, H_OUT * UP), jnp.float32),
        pltpu.SemaphoreType.DMA,
        pltpu.SemaphoreType.DMA,
        pltpu.SemaphoreType.DMA,
        pltpu.SemaphoreType.DMA,
    ],
)
def _mulut_main(table_hbm, img_hbm, out_hbm, img_v, idx_v, coef_v, rows_v,
                out_v, sem0, sem1, sem_img, sem_out):
    wid = _worker_id()
    iota = lax.broadcasted_iota(jnp.int32, (16,), 0)
    sems = (sem0, sem1)
    NROWS = 8 * CHUNK
    mask_hi = jnp.full((16,), -65536, jnp.int32)

    def img_copy(t):
        src_row = (t >> 8) * H_IN + (t & 255)
        return pltpu.make_async_copy(img_hbm.at[pl.ds(src_row, 2)], img_v,
                                     sem_img)

    def out_copy(t):
        return pltpu.make_async_copy(out_v, out_hbm.at[pl.ds(t * UP, UP)],
                                     sem_out)

    def gen(ch, buf):
        p0 = ch * CHUNK

        def gen_body(g, _):
            o = p0 + g * 16
            a = img_v[0, pl.ds(o, 16)]
            b = img_v[0, pl.ds(o + 1, 16)]
            c = img_v[1, pl.ds(o, 16)]
            d = img_v[1, pl.ds(o + 1, 16)]
            base = ((a >> 4) * 4913 + (b >> 4) * 289
                    + (c >> 4) * 17 + (d >> 4))
            fa = (a & 15).astype(jnp.float32)
            fb = (b & 15).astype(jnp.float32)
            fc = (c & 15).astype(jnp.float32)
            fd = (d & 15).astype(jnp.float32)
            ta = (16.0 - fa, fa)
            tb = (16.0 - fb, fb)
            tc = (16.0 - fc, fc)
            td = (16.0 - fd, fd)
            pab = [[ta[x] * tb[y] for y in range(2)] for x in range(2)]
            pcd = [[tc[x] * td[y] for y in range(2)] for x in range(2)]
            col = pl.ds(g * 16, 16)
            for p in range(8):
                idx_v[buf * 8 + p, col] = base + _PAIR_OFF[p]
            for m in range(16):
                b0, b1, b2, b3 = (m >> 3) & 1, (m >> 2) & 1, (m >> 1) & 1, m & 1
                coef_v[buf * 16 + m, col] = pab[b0][b1] * pcd[b2][b3]
            return 0

        lax.fori_loop(0, CHUNK // 16, gen_body, 0)

    def fire(buf):
        for p in range(8):
            pltpu.async_copy(
                table_hbm.at[idx_v.at[buf * 8 + p]],
                rows_v.at[pl.ds(buf * NROWS + p * CHUNK, CHUNK)],
                sems[buf])

    def drain(buf):
        pltpu.make_async_copy(table_hbm.at[pl.ds(0, NROWS)],
                              rows_v.at[pl.ds(buf * NROWS, NROWS)],
                              sems[buf]).wait()

    def acc(ch, buf):
        p0 = ch * CHUNK

        dnums = lax.GatherDimensionNumbers(offset_dims=(),
                                           collapsed_slice_dims=(0,),
                                           start_index_map=(0,))

        def bcast(v, m):
            return lax.gather(v, jnp.full((16, 1), m, jnp.int32), dnums,
                              (1,), mode=lax.GatherScatterMode.PROMISE_IN_BOUNDS)

        def acc_body(jg, _):
            col = pl.ds(jg * 16, 16)
            cl = [coef_v[buf * 16 + m, col] for m in range(16)]
            orow = lax.shift_right_logical(iota, 2)
            for u in range(16):
                j = jg * 16 + u
                accv = None
                for p in range(8):
                    xr = rows_v[buf * NROWS + p * CHUNK + j, :]
                    vlo = plsc.bitcast(lax.shift_left(xr, 16), jnp.float32)
                    vhi = plsc.bitcast(xr & mask_hi, jnp.float32)
                    t = bcast(cl[2 * p], u) * vlo + bcast(cl[2 * p + 1], u) * vhi
                    accv = t if p == 0 else accv + t
                plsc.store_scatter(
                    out_v, [orow, (p0 + j) * UP + (iota & 3)],
                    accv * INV_SCALE)
            return 0

        lax.fori_loop(0, CHUNK // 16, acc_body, 0)

    t0 = wid * ROWS_PER_WORKER
    img_copy(t0).start()
    img_copy(t0).wait()
    gen(0, 0)
    fire(0)
    gen(1, 1)
    fire(1)

    def row_body(i, _):
        t = t0 + i

        @pl.when(i < ROWS_PER_WORKER - 1)
        def _():
            img_copy(t + 1).start()

        drain(0)

        @pl.when(i > 0)
        def _():
            out_copy(t - 1).wait()

        acc(0, 0)

        @pl.when(i < ROWS_PER_WORKER - 1)
        def _():
            img_copy(t + 1).wait()
            gen(0, 0)
            fire(0)

        drain(1)
        acc(1, 1)

        @pl.when(i < ROWS_PER_WORKER - 1)
        def _():
            gen(1, 1)
            fire(1)

        out_copy(t).start()
        return 0

    lax.fori_loop(0, ROWS_PER_WORKER, row_body, 0)
    out_copy(t0 + ROWS_PER_WORKER - 1).wait()


def kernel(img_in, weight):
    B, C, H, W = img_in.shape
    img2 = img_in.reshape(B * C * H, W)
    img2 = jnp.pad(img2, ((0, 0), (0, W_PAD - W)))
    wflat = weight.reshape(-1)
    wpad = jnp.pad(wflat, (0, W_QPAD - wflat.shape[0]))
    table = _quantize_lut(wpad).reshape(TABLE_ROWS, 16)
    out = _mulut_main(table, img2)
    return out.reshape(B, C, H_OUT * UP, H_OUT * UP)
